# baseline (speedup 1.0000x reference)
import functools

import jax
import jax.numpy as jnp
from jax import lax
from jax.experimental import pallas as pl
from jax.experimental.pallas import tpu as pltpu
from jax.experimental.pallas import tpu_sc as plsc

_DICT = 1000
_D = 128
_H = 256
_B = 1024
_S = 50
_SPAD = 56
_NW = 32
_NCHUNK = 14
_CK = 128
_ROWS_W = _NCHUNK * _CK

_BB = 128
_SB = 8
_NB = _B // _BB
_NS = _SPAD // _SB


def _sc_gather(emb, idx3):
    mesh = plsc.VectorSubcoreMesh(core_axis_name="c", subcore_axis_name="s")

    @functools.partial(
        pl.kernel,
        out_type=jax.ShapeDtypeStruct((_NW * _NCHUNK, _CK, _D), jnp.float32),
        mesh=mesh,
        scratch_types=[
            pltpu.VMEM((_NCHUNK, _CK), jnp.int32),
            pltpu.VMEM((_NCHUNK // 2, _CK, _D), jnp.float32),
            pltpu.VMEM_SHARED((_DICT, _D), jnp.float32),
            pltpu.SemaphoreType.DMA,
        ],
    )
    def k(table_hbm, idx_hbm, out_hbm, idx_v, buf, tab_s, gsem):
        sid = lax.axis_index("s")
        wid = sid * 2 + lax.axis_index("c")
        half = _NCHUNK // 2

        @pl.when(sid == 0)
        def _():
            pltpu.sync_copy(table_hbm, tab_s)

        pltpu.sync_copy(idx_hbm.at[wid], idx_v)
        plsc.subcore_barrier()
        for j in range(2):
            cps = [
                pltpu.async_copy(
                    tab_s.at[idx_v.at[j * half + c]], buf.at[c], gsem
                )
                for c in range(half)
            ]
            for cp in cps:
                cp.wait()
            pltpu.sync_copy(buf, out_hbm.at[pl.ds(wid * _NCHUNK + j * half, half)])

    return k(emb, idx3)


def _sigmoid(x):
    return 0.5 * jnp.tanh(0.5 * x) + 0.5


def _lstm_proj_body(e_ref, w_ref, u_ref, b_ref, wfc_ref, bfc_ref, out_ref,
                    h_ref, c_ref, hs_ref):
    s_blk = pl.program_id(1)

    @pl.when(s_blk == 0)
    def _():
        h_ref[...] = jnp.zeros_like(h_ref)
        c_ref[...] = jnp.zeros_like(c_ref)

    e = e_ref[...].reshape(_BB * _SB, _D)
    zx = jnp.dot(e, w_ref[...], preferred_element_type=jnp.float32)
    zx = zx + b_ref[...]

    h = h_ref[...]
    c = c_ref[...]
    u = u_ref[...]
    for t in range(_SB):
        z = zx[t * _BB:(t + 1) * _BB, :] + jnp.dot(
            h, u, preferred_element_type=jnp.float32)
        i = _sigmoid(z[:, 0 * _H:1 * _H])
        f = _sigmoid(z[:, 1 * _H:2 * _H])
        g = jnp.tanh(z[:, 2 * _H:3 * _H])
        o = _sigmoid(z[:, 3 * _H:4 * _H])
        c = f * c + i * g
        h = o * jnp.tanh(c)
        hs_ref[:, t, :] = h
    h_ref[...] = h
    c_ref[...] = c

    hs = hs_ref[...].reshape(_BB * _SB, _H)
    out = jnp.dot(hs, wfc_ref[...], preferred_element_type=jnp.float32)
    out = out + bfc_ref[...]
    out_ref[...] = out.reshape(_BB, _SB, _DICT)


def _lstm_proj(e3, w, u, b2, wfc, bfc2):
    return pl.pallas_call(
        _lstm_proj_body,
        grid=(_NB, _NS),
        in_specs=[
            pl.BlockSpec((1, _BB * _SB, _D), lambda ib, s: (ib * _NS + s, 0, 0)),
            pl.BlockSpec((_D, 4 * _H), lambda ib, s: (0, 0)),
            pl.BlockSpec((_H, 4 * _H), lambda ib, s: (0, 0)),
            pl.BlockSpec((1, 4 * _H), lambda ib, s: (0, 0)),
            pl.BlockSpec((_H, _DICT), lambda ib, s: (0, 0)),
            pl.BlockSpec((1, _DICT), lambda ib, s: (0, 0)),
        ],
        out_specs=pl.BlockSpec((_BB, _SB, _DICT), lambda ib, s: (ib, s, 0)),
        out_shape=jax.ShapeDtypeStruct((_B, _S, _DICT), jnp.float32),
        scratch_shapes=[
            pltpu.VMEM((_BB, _H), jnp.float32),
            pltpu.VMEM((_BB, _H), jnp.float32),
            pltpu.VMEM((_BB, _SB, _H), jnp.float32),
        ],
        compiler_params=pltpu.CompilerParams(
            dimension_semantics=("parallel", "arbitrary"),
        ),
    )(e3, w, u, b2, wfc, bfc2)


def kernel(x, emb, W_lstm, U_lstm, b_lstm, W_fc, b_fc):
    x = x.astype(jnp.int32)
    xpad = jnp.pad(x, ((0, 0), (0, _SPAD - _S)))
    perm = xpad.reshape(_NB, _BB, _NS, _SB).transpose(0, 2, 3, 1)
    idx3 = perm.reshape(_NW, _NCHUNK, _CK)
    e_flat = _sc_gather(emb, idx3)
    e3 = e_flat.reshape(_NB * _NS, _BB * _SB, _D)
    return _lstm_proj(
        e3, W_lstm, U_lstm, b_lstm.reshape(1, 4 * _H),
        W_fc, b_fc.reshape(1, _DICT),
    )


# speedup vs baseline: 1.0849x; 1.0849x over previous
import functools

import jax
import jax.numpy as jnp
from jax import lax
from jax.experimental import pallas as pl
from jax.experimental.pallas import tpu as pltpu
from jax.experimental.pallas import tpu_sc as plsc

_DICT = 1000
_D = 128
_H = 256
_B = 1024
_S = 50
_SPAD = 56
_NW = 32
_NCHUNK = 14
_CK = 128
_ROWS_W = _NCHUNK * _CK

_BB = 256
_SB = 8
_NB = _B // _BB
_NS = _SPAD // _SB


def _sc_gather(emb, idx3):
    mesh = plsc.VectorSubcoreMesh(core_axis_name="c", subcore_axis_name="s")

    @functools.partial(
        pl.kernel,
        out_type=jax.ShapeDtypeStruct((_NW * _NCHUNK, _CK, _D), jnp.float32),
        mesh=mesh,
        scratch_types=[
            pltpu.VMEM((_NCHUNK, _CK), jnp.int32),
            pltpu.VMEM((_NCHUNK // 2, _CK, _D), jnp.float32),
            pltpu.VMEM_SHARED((_DICT, _D), jnp.float32),
            pltpu.SemaphoreType.DMA,
        ],
    )
    def k(table_hbm, idx_hbm, out_hbm, idx_v, buf, tab_s, gsem):
        sid = lax.axis_index("s")
        wid = sid * 2 + lax.axis_index("c")
        half = _NCHUNK // 2

        @pl.when(sid == 0)
        def _():
            pltpu.sync_copy(table_hbm, tab_s)

        pltpu.sync_copy(idx_hbm.at[wid], idx_v)
        plsc.subcore_barrier()
        for j in range(2):
            cps = [
                pltpu.async_copy(
                    tab_s.at[idx_v.at[j * half + c]], buf.at[c], gsem
                )
                for c in range(half)
            ]
            for cp in cps:
                cp.wait()
            pltpu.sync_copy(buf, out_hbm.at[pl.ds(wid * _NCHUNK + j * half, half)])

    return k(emb, idx3)


def _sigmoid(x):
    return 0.5 * jnp.tanh(0.5 * x) + 0.5


_NOB = 3
_SREM = _S - (_NS - 1) * _SB


def _out_copy(obuf, out_hbm, osem, ph, ib, s, partial):
    if isinstance(s, int):
        s_c = min(s, _NS - 2) if not partial else s
    else:
        s_c = jnp.minimum(s, _NS - 2) if not partial else s
    ib = max(ib, 0) if isinstance(ib, int) else jnp.maximum(ib, 0)
    n = _SREM if partial else _SB
    src = obuf.at[ph, :, 0:_SREM, :] if partial else obuf.at[ph]
    return pltpu.make_async_copy(
        src,
        out_hbm.at[pl.ds(ib * _BB, _BB), pl.ds(s_c * _SB, n), :],
        osem.at[ph])


def _lstm_proj_body(e_ref, w_ref, u_ref, b_ref, wfc_ref, bfc_ref, out_hbm,
                    h_ref, c_ref, hs_ref, obuf, osem):
    ib = pl.program_id(0)
    s = pl.program_id(1)
    k = ib * _NS + s
    ph = lax.rem(k, _NOB)

    @pl.when(s == 0)
    def _():
        h_ref[...] = jnp.zeros_like(h_ref)
        c_ref[...] = jnp.zeros_like(c_ref)

    kp = k - _NOB
    ibp = kp // _NS
    sp = kp - ibp * _NS

    @pl.when(jnp.logical_and(kp >= 0, sp < _NS - 1))
    def _():
        _out_copy(obuf, out_hbm, osem, ph, ibp, sp, False).wait()

    @pl.when(jnp.logical_and(kp >= 0, sp == _NS - 1))
    def _():
        _out_copy(obuf, out_hbm, osem, ph, ibp, sp, True).wait()

    e = e_ref[...].reshape(_BB * _SB, _D)
    zx = jnp.dot(e, w_ref[...], preferred_element_type=jnp.float32)
    zx = zx + b_ref[...]

    h = h_ref[...]
    c = c_ref[...]
    u = u_ref[...]
    for t in range(_SB):
        z = zx[t * _BB:(t + 1) * _BB, :] + jnp.dot(
            h, u, preferred_element_type=jnp.float32)
        i = _sigmoid(z[:, 0 * _H:1 * _H])
        f = _sigmoid(z[:, 1 * _H:2 * _H])
        g = jnp.tanh(z[:, 2 * _H:3 * _H])
        o = _sigmoid(z[:, 3 * _H:4 * _H])
        c = f * c + i * g
        h = o * jnp.tanh(c)
        hs_ref[:, t, :] = h
    h_ref[...] = h
    c_ref[...] = c

    hs = hs_ref[...].reshape(_BB * _SB, _H)
    out = jnp.dot(hs, wfc_ref[...], preferred_element_type=jnp.float32)
    out = out + bfc_ref[...]
    obuf[ph] = out.reshape(_BB, _SB, _DICT)

    @pl.when(s < _NS - 1)
    def _():
        _out_copy(obuf, out_hbm, osem, ph, ib, s, False).start()

    @pl.when(s == _NS - 1)
    def _():
        _out_copy(obuf, out_hbm, osem, ph, ib, s, True).start()

    if True:
        k_last = _NB * _NS - 1

        @pl.when(k == k_last)
        def _():
            for d in range(_NOB):
                kq = k_last - d
                ibq, sq = kq // _NS, kq % _NS
                _out_copy(obuf, out_hbm, osem, kq % _NOB, ibq, sq,
                          sq == _NS - 1).wait()


def _lstm_proj(e3, w, u, b2, wfc, bfc2):
    return pl.pallas_call(
        _lstm_proj_body,
        grid=(_NB, _NS),
        in_specs=[
            pl.BlockSpec((1, _BB * _SB, _D), lambda ib, s: (ib * _NS + s, 0, 0)),
            pl.BlockSpec((_D, 4 * _H), lambda ib, s: (0, 0)),
            pl.BlockSpec((_H, 4 * _H), lambda ib, s: (0, 0)),
            pl.BlockSpec((1, 4 * _H), lambda ib, s: (0, 0)),
            pl.BlockSpec((_H, _DICT), lambda ib, s: (0, 0)),
            pl.BlockSpec((1, _DICT), lambda ib, s: (0, 0)),
        ],
        out_specs=pl.BlockSpec(memory_space=pltpu.MemorySpace.HBM),
        out_shape=jax.ShapeDtypeStruct((_B, _S, _DICT), jnp.float32),
        scratch_shapes=[
            pltpu.VMEM((_BB, _H), jnp.float32),
            pltpu.VMEM((_BB, _H), jnp.float32),
            pltpu.VMEM((_BB, _SB, _H), jnp.float32),
            pltpu.VMEM((_NOB, _BB, _SB, _DICT), jnp.float32),
            pltpu.SemaphoreType.DMA((_NOB,)),
        ],
        compiler_params=pltpu.CompilerParams(
            dimension_semantics=("arbitrary", "arbitrary"),
        ),
    )(e3, w, u, b2, wfc, bfc2)


def kernel(x, emb, W_lstm, U_lstm, b_lstm, W_fc, b_fc):
    x = x.astype(jnp.int32)
    xpad = jnp.pad(x, ((0, 0), (0, _SPAD - _S)))
    perm = xpad.reshape(_NB, _BB, _NS, _SB).transpose(0, 2, 3, 1)
    idx3 = perm.reshape(_NW, _NCHUNK, _CK)
    e_flat = _sc_gather(emb, idx3)
    e3 = e_flat.reshape(_NB * _NS, _BB * _SB, _D)
    return _lstm_proj(
        e3, W_lstm, U_lstm, b_lstm.reshape(1, 4 * _H),
        W_fc, b_fc.reshape(1, _DICT),
    )
